# baseline (speedup 1.0000x reference)
import functools

import jax
import jax.numpy as jnp
import numpy as np
from jax import lax
from jax.experimental import pallas as pl
from jax.experimental.pallas import tpu as pltpu
from jax.experimental.pallas import tpu_sc as plsc

_V = 100000
_S = 2048
_D = 1024
_B = 4

_NC, _NS = 2, 16
_NW = _NC * _NS
_POS_PER_W = _S // _NW
_PC = 8
_NSTEPS = _POS_PER_W // _PC
_ROWS = _PC * _B
_LANES = 16


def _positional_encoding(seq: int, d: int) -> jnp.ndarray:
    pos = np.arange(seq, dtype=np.float32)[:, None]
    i = np.arange(d, dtype=np.float32)[None, :]
    ang = pos / np.power(10000.0, (2.0 * np.floor(i / 2.0)) / float(d))
    pe = np.zeros((seq, d), dtype=np.float32)
    pe[:, 0::2] = np.sin(ang[:, 0::2])
    pe[:, 1::2] = np.cos(ang[:, 1::2])
    return jnp.asarray(pe)


def _add_pe_group(rows_v, pe_v, bs):
    @plsc.parallel_loop(0, _PC, 1)
    def _(r):
        for c in range(0, _D, _LANES):
            v = pe_v[r, pl.ds(c, _LANES)]
            for b in bs:
                plsc.addupdate(rows_v.at[b * _PC + r, pl.ds(c, _LANES)], v)


_NBUF = 3


def _body(x_hbm, table_hbm, pe_hbm, out_hbm,
          idx_v, pe_0, pe_1, pe_2, rows_0, rows_1, rows_2,
          g_sem_0, g_sem_1, g_sem_2, p_sem_0, p_sem_1, p_sem_2,
          s_sem_0, s_sem_1, s_sem_2):
    wid = lax.axis_index("s") * _NC + lax.axis_index("c")
    pos0 = wid * _POS_PER_W

    row_bufs = (rows_0, rows_1, rows_2)
    pe_bufs = (pe_0, pe_1, pe_2)
    g_sems = (g_sem_0, g_sem_1, g_sem_2)
    p_sems = (p_sem_0, p_sem_1, p_sem_2)
    s_sems = (s_sem_0, s_sem_1, s_sem_2)

    idx_copies = [
        pltpu.async_copy(x_hbm.at[pl.ds(b * _S + pos0, _POS_PER_W)],
                         idx_v.at[pl.ds(b * _POS_PER_W, _POS_PER_W)],
                         g_sem_0)
        for b in range(_B)
    ]

    def start_pe(s):
        k = s % _NBUF
        return pltpu.async_copy(pe_hbm.at[pl.ds(pos0 + s * _PC, _PC)],
                                pe_bufs[k], p_sems[k])

    def start_gathers(s):
        k = s % _NBUF
        gs = []
        for b in range(_B):
            idx = idx_v.at[pl.ds(b * _POS_PER_W + s * _PC, _PC)]
            gs.append(pltpu.async_copy(
                table_hbm.at[idx],
                row_bufs[k].at[pl.ds(b * _PC, _PC)], g_sems[k]))
        return gs

    def start_step(s):
        return start_gathers(s), start_pe(s)

    _primed = min(_NBUF - 1, _NSTEPS)
    pe_inflight = [start_pe(s) for s in range(_primed)]
    for c in idx_copies:
        c.wait()
    inflight = {s: (start_gathers(s), pe_inflight[s]) for s in range(_primed)}
    stores = {}
    for s in range(_NSTEPS):
        k = s % _NBUF
        gs, p = inflight[s]
        for g in gs:
            g.wait()
        if s - 1 in stores:
            for st in stores[s - 1]:
                st.wait()
        if s + _NBUF - 1 < _NSTEPS:
            inflight[s + _NBUF - 1] = start_step(s + _NBUF - 1)
        p.wait()
        stores[s] = []
        for bs in ((0, 1), (2, 3)):
            _add_pe_group(row_bufs[k], pe_bufs[k], bs)
            for b in bs:
                flat = b * _S + pos0 + s * _PC
                stores[s].append(pltpu.async_copy(
                    row_bufs[k].at[pl.ds(b * _PC, _PC)],
                    out_hbm.at[pl.ds(flat, _PC)], s_sems[k]))
    for st in stores[_NSTEPS - 1]:
        st.wait()


@jax.jit
def _run(x_flat, table, pe):
    mesh = plsc.VectorSubcoreMesh(
        core_axis_name="c", subcore_axis_name="s",
        num_cores=_NC, num_subcores=_NS,
    )
    f = pl.kernel(
        _body,
        out_type=jax.ShapeDtypeStruct((_B * _S, _D), jnp.float32),
        mesh=mesh,
        scratch_types=(
            [pltpu.VMEM((_B * _POS_PER_W,), jnp.int32)]
            + [pltpu.VMEM((_PC, _D), jnp.float32)] * _NBUF
            + [pltpu.VMEM((_ROWS, _D), jnp.float32)] * _NBUF
            + [pltpu.SemaphoreType.DMA] * (3 * _NBUF)
        ),
    )
    return f(x_flat, table, pe)


def kernel(x, table):
    pe = _positional_encoding(_S, _D)
    out = _run(x.astype(jnp.int32).reshape(-1), table, pe)
    return out.reshape(_B, _S, _D)


# speedup vs baseline: 1.0429x; 1.0429x over previous
import functools

import jax
import jax.numpy as jnp
import numpy as np
from jax import lax
from jax.experimental import pallas as pl
from jax.experimental.pallas import tpu as pltpu
from jax.experimental.pallas import tpu_sc as plsc

_V = 100000
_S = 2048
_D = 1024
_B = 4

_NC, _NS = 2, 16
_NW = _NC * _NS
_POS_PER_W = _S // _NW
_PC = 8
_NSTEPS = _POS_PER_W // _PC
_ROWS = _PC * _B
_LANES = 16


def _positional_encoding(seq: int, d: int) -> jnp.ndarray:
    pos = np.arange(seq, dtype=np.float32)[:, None]
    i = np.arange(d, dtype=np.float32)[None, :]
    ang = pos / np.power(10000.0, (2.0 * np.floor(i / 2.0)) / float(d))
    pe = np.zeros((seq, d), dtype=np.float32)
    pe[:, 0::2] = np.sin(ang[:, 0::2])
    pe[:, 1::2] = np.cos(ang[:, 1::2])
    return jnp.asarray(pe)


def _add_pe(rows_v, pe_v):
    @plsc.parallel_loop(0, _PC, 1)
    def _(r):
        for c in range(0, _D, _LANES):
            v = pe_v[r, pl.ds(c, _LANES)]
            for b in range(_B):
                plsc.addupdate(rows_v.at[b * _PC + r, pl.ds(c, _LANES)], v)


_NBUF = 3


def _body(x_hbm, table_hbm, pe_hbm, out_hbm,
          idx_v, pe_0, pe_1, pe_2, rows_0, rows_1, rows_2,
          g_sem_0, g_sem_1, g_sem_2, p_sem_0, p_sem_1, p_sem_2,
          s_sem_0, s_sem_1, s_sem_2):
    wid = lax.axis_index("s") * _NC + lax.axis_index("c")
    pos0 = wid * _POS_PER_W

    row_bufs = (rows_0, rows_1, rows_2)
    pe_bufs = (pe_0, pe_1, pe_2)
    g_sems = (g_sem_0, g_sem_1, g_sem_2)
    p_sems = (p_sem_0, p_sem_1, p_sem_2)
    s_sems = (s_sem_0, s_sem_1, s_sem_2)

    idx_copies = [
        pltpu.async_copy(x_hbm.at[b, pl.ds(pos0, _POS_PER_W)],
                         idx_v.at[pl.ds(b * _POS_PER_W, _POS_PER_W)],
                         g_sem_0)
        for b in range(_B)
    ]

    def start_pe(s):
        k = s % _NBUF
        return pltpu.async_copy(pe_hbm.at[pl.ds(pos0 + s * _PC, _PC)],
                                pe_bufs[k], p_sems[k])

    def start_gathers(s):
        k = s % _NBUF
        gs = []
        for b in range(_B):
            idx = idx_v.at[pl.ds(b * _POS_PER_W + s * _PC, _PC)]
            gs.append(pltpu.async_copy(
                table_hbm.at[idx],
                row_bufs[k].at[pl.ds(b * _PC, _PC)], g_sems[k]))
        return gs

    def start_step(s):
        return start_gathers(s), start_pe(s)

    _primed = min(_NBUF - 1, _NSTEPS)
    pe_inflight = [start_pe(s) for s in range(_primed)]
    for c in idx_copies:
        c.wait()
    inflight = {s: (start_gathers(s), pe_inflight[s]) for s in range(_primed)}
    stores = {}
    for s in range(_NSTEPS):
        k = s % _NBUF
        gs, p = inflight[s]
        for g in gs:
            g.wait()
        if s - 1 in stores:
            for st in stores[s - 1]:
                st.wait()
        if s + _NBUF - 1 < _NSTEPS:
            inflight[s + _NBUF - 1] = start_step(s + _NBUF - 1)
        p.wait()
        _add_pe(row_bufs[k], pe_bufs[k])
        stores[s] = []
        for b in range(_B):
            flat = b * _S + pos0 + s * _PC
            stores[s].append(pltpu.async_copy(
                row_bufs[k].at[pl.ds(b * _PC, _PC)],
                out_hbm.at[pl.ds(flat, _PC)], s_sems[k]))
    for st in stores[_NSTEPS - 1]:
        st.wait()


@jax.jit
def _run(x_flat, table, pe):
    mesh = plsc.VectorSubcoreMesh(
        core_axis_name="c", subcore_axis_name="s",
        num_cores=_NC, num_subcores=_NS,
    )
    f = pl.kernel(
        _body,
        out_type=jax.ShapeDtypeStruct((_B * _S, _D), jnp.float32),
        mesh=mesh,
        scratch_types=(
            [pltpu.VMEM((_B * _POS_PER_W,), jnp.int32)]
            + [pltpu.VMEM((_PC, _D), jnp.float32)] * _NBUF
            + [pltpu.VMEM((_ROWS, _D), jnp.float32)] * _NBUF
            + [pltpu.SemaphoreType.DMA] * (3 * _NBUF)
        ),
    )
    return f(x_flat, table, pe)


def kernel(x, table):
    pe = _positional_encoding(_S, _D)
    out = _run(x.astype(jnp.int32), table, pe)
    return out.reshape(_B, _S, _D)


# speedup vs baseline: 1.0484x; 1.0053x over previous
import functools

import jax
import jax.numpy as jnp
import numpy as np
from jax import lax
from jax.experimental import pallas as pl
from jax.experimental.pallas import tpu as pltpu
from jax.experimental.pallas import tpu_sc as plsc

_V = 100000
_S = 2048
_D = 1024
_B = 4

_NC, _NS = 2, 16
_NW = _NC * _NS
_POS_PER_W = _S // _NW
_PC = 8
_NSTEPS = _POS_PER_W // _PC
_ROWS = _PC * _B
_LANES = 16


def _positional_encoding(seq: int, d: int) -> jnp.ndarray:
    pos = np.arange(seq, dtype=np.float32)[:, None]
    i = np.arange(d, dtype=np.float32)[None, :]
    ang = pos / np.power(10000.0, (2.0 * np.floor(i / 2.0)) / float(d))
    pe = np.zeros((seq, d), dtype=np.float32)
    pe[:, 0::2] = np.sin(ang[:, 0::2])
    pe[:, 1::2] = np.cos(ang[:, 1::2])
    return jnp.asarray(pe)


def _add_pe(rows_v, pe_v):
    @plsc.parallel_loop(0, _PC, 1)
    def _(r):
        for c in range(0, _D, _LANES):
            v = pe_v[r, pl.ds(c, _LANES)]
            for b in range(_B):
                plsc.addupdate(rows_v.at[b * _PC + r, pl.ds(c, _LANES)], v)


_NBUF = 3


def _body(x_hbm, table_hbm, pe_hbm, out_hbm,
          idx_v, pe_0, pe_1, pe_2, rows_0, rows_1, rows_2,
          g_sem_0, g_sem_1, g_sem_2, p_sem_0, p_sem_1, p_sem_2,
          s_sem_0, s_sem_1, s_sem_2):
    wid = lax.axis_index("s") * _NC + lax.axis_index("c")
    pos0 = wid * _POS_PER_W

    row_bufs = (rows_0, rows_1, rows_2)
    pe_bufs = (pe_0, pe_1, pe_2)
    g_sems = (g_sem_0, g_sem_1, g_sem_2)
    p_sems = (p_sem_0, p_sem_1, p_sem_2)
    s_sems = (s_sem_0, s_sem_1, s_sem_2)

    idx_copies = [
        pltpu.async_copy(x_hbm.at[b, pl.ds(pos0, _POS_PER_W)],
                         idx_v.at[pl.ds(b * _POS_PER_W, _POS_PER_W)],
                         g_sem_0)
        for b in range(_B)
    ]

    def start_pe(s):
        k = s % _NBUF
        return pltpu.async_copy(pe_hbm.at[pl.ds(pos0 + s * _PC, _PC)],
                                pe_bufs[k], p_sems[k])

    def start_gathers(s):
        k = s % _NBUF
        gs = []
        for b in range(_B):
            idx = idx_v.at[pl.ds(b * _POS_PER_W + s * _PC, _PC)]
            gs.append(pltpu.async_copy(
                table_hbm.at[idx],
                row_bufs[k].at[pl.ds(b * _PC, _PC)], g_sems[k],
                priority=1))
        return gs

    def start_step(s):
        return start_gathers(s), start_pe(s)

    _primed = min(_NBUF - 1, _NSTEPS)
    pe_inflight = [start_pe(s) for s in range(_primed)]
    for c in idx_copies:
        c.wait()
    inflight = {s: (start_gathers(s), pe_inflight[s]) for s in range(_primed)}
    stores = {}
    for s in range(_NSTEPS):
        k = s % _NBUF
        gs, p = inflight[s]
        for g in gs:
            g.wait()
        if s - 1 in stores:
            for st in stores[s - 1]:
                st.wait()
        if s + _NBUF - 1 < _NSTEPS:
            inflight[s + _NBUF - 1] = start_step(s + _NBUF - 1)
        p.wait()
        _add_pe(row_bufs[k], pe_bufs[k])
        stores[s] = []
        for b in range(_B):
            flat = b * _S + pos0 + s * _PC
            stores[s].append(pltpu.async_copy(
                row_bufs[k].at[pl.ds(b * _PC, _PC)],
                out_hbm.at[pl.ds(flat, _PC)], s_sems[k]))
    for st in stores[_NSTEPS - 1]:
        st.wait()


@jax.jit
def _run(x_flat, table, pe):
    mesh = plsc.VectorSubcoreMesh(
        core_axis_name="c", subcore_axis_name="s",
        num_cores=_NC, num_subcores=_NS,
    )
    f = pl.kernel(
        _body,
        out_type=jax.ShapeDtypeStruct((_B * _S, _D), jnp.float32),
        mesh=mesh,
        scratch_types=(
            [pltpu.VMEM((_B * _POS_PER_W,), jnp.int32)]
            + [pltpu.VMEM((_PC, _D), jnp.float32)] * _NBUF
            + [pltpu.VMEM((_ROWS, _D), jnp.float32)] * _NBUF
            + [pltpu.SemaphoreType.DMA] * (3 * _NBUF)
        ),
    )
    return f(x_flat, table, pe)


def kernel(x, table):
    pe = _positional_encoding(_S, _D)
    out = _run(x.astype(jnp.int32), table, pe)
    return out.reshape(_B, _S, _D)
